# Initial kernel scaffold; baseline (speedup 1.0000x reference)
import jax
import jax.numpy as jnp
from jax.experimental import pallas as pl


def kernel(logits, temperatures, exp_noise):
    raise NotImplementedError("write your pallas kernel here")


# speedup vs baseline: 3.1367x; 3.1367x over previous
import jax
import jax.numpy as jnp
from jax.experimental import pallas as pl
from jax.experimental.pallas import tpu as pltpu

_B = 64
_V = 1000000
_VC = 32768
_NCHUNK = (_V + _VC - 1) // _VC


def _tc_body(temp_ref, logits_ref, noise_ref, out_ref, best_val, best_idx):
    i = pl.program_id(0)
    temp = temp_ref[...]
    logits = logits_ref[...]
    noise = noise_ref[...]
    col = jax.lax.broadcasted_iota(jnp.int32, (_B, _VC), 1) + i * _VC
    valid = col < _V
    key = logits / temp - jnp.log(noise + 1e-10)
    key = jnp.where(temp == 0.0, logits, key)
    key = jnp.where(valid, key, -jnp.inf)
    local_max = jnp.max(key, axis=1, keepdims=True)
    at_max = (key == local_max) & valid
    local_idx = jnp.min(jnp.where(at_max, col, _V), axis=1, keepdims=True)

    @pl.when(i == 0)
    def _():
        best_val[...] = local_max
        best_idx[...] = local_idx

    @pl.when(i > 0)
    def _():
        bv = best_val[...]
        take = local_max > bv
        best_val[...] = jnp.where(take, local_max, bv)
        best_idx[...] = jnp.where(take, local_idx, best_idx[...])

    @pl.when(i == _NCHUNK - 1)
    def _():
        out_ref[...] = best_idx[...]


@jax.jit
def kernel(logits, temperatures, exp_noise):
    temps = temperatures.astype(jnp.float32).reshape(_B, 1)
    out = pl.pallas_call(
        _tc_body,
        grid=(_NCHUNK,),
        in_specs=[
            pl.BlockSpec((_B, 1), lambda i: (0, 0)),
            pl.BlockSpec((_B, _VC), lambda i: (0, i)),
            pl.BlockSpec((_B, _VC), lambda i: (0, i)),
        ],
        out_specs=pl.BlockSpec((_B, 1), lambda i: (0, 0)),
        out_shape=jax.ShapeDtypeStruct((_B, 1), jnp.int32),
        scratch_shapes=[
            pltpu.VMEM((_B, 1), jnp.float32),
            pltpu.VMEM((_B, 1), jnp.int32),
        ],
    )(temps, logits.astype(jnp.float32), exp_noise)
    return out.reshape(_B)
